# baseline (speedup 1.0000x reference)
import functools

import jax
import jax.numpy as jnp
from jax import lax
from jax.experimental import pallas as pl
from jax.experimental.pallas import tpu as pltpu
from jax.experimental.pallas import tpu_sc as plsc

DP = 16
NC, NS, LN = 2, 16, 16


@functools.cache
def _make_sc(B, L, H):
    NW = NC * NS
    rows_per_w = B // NW
    G = 16
    CH = G * L
    n_chunks = rows_per_w // G
    assert B % NW == 0 and rows_per_w % (2 * G) == 0 and L % 8 == 0
    assert H == 2 * LN
    mesh = plsc.VectorSubcoreMesh(core_axis_name="c", subcore_axis_name="s")

    @functools.partial(
        pl.kernel,
        out_type=jax.ShapeDtypeStruct((B // 4, 4 * H), jnp.float32),
        mesh=mesh,
        compiler_params=pltpu.CompilerParams(use_tc_tiling_on_sc=False,
                                             needs_layout_passes=False),
        scratch_types=[
            pltpu.VMEM((CH,), jnp.int32),
            pltpu.VMEM((CH,), jnp.int32),
            pltpu.VMEM((CH, H), jnp.bfloat16),
            pltpu.VMEM((CH, H), jnp.bfloat16),
            pltpu.VMEM((G // 4, 4 * H), jnp.float32),
            pltpu.SemaphoreType.DMA,
            pltpu.SemaphoreType.DMA,
        ],
    )
    def sc(text_hbm, emb_hbm, out_hbm, idx_a, idx_b, rows_a, rows_b,
           pooled_v, sem_a, sem_b):
        wid = lax.axis_index("s") * NC + lax.axis_index("c")
        base_row = wid * rows_per_w

        def fetch(c, idx_v, sem, rows_v):
            pltpu.sync_copy(text_hbm.at[pl.ds((base_row + c * G) * L, CH)],
                            idx_v)
            pltpu.async_copy(emb_hbm.at[idx_v], rows_v, sem)

        def drain_and_acc(c, idx_v, sem, rows_v):
            pltpu.make_async_copy(emb_hbm.at[idx_v], rows_v, sem).wait()

            def row_body(r, _):
                def acc_body(i, carry):
                    a0, a1 = carry
                    t = r * L + i * 8
                    for u in range(8):
                        ev, od = plsc.unpack(
                            rows_v[t + u], format=plsc.PackFormat.INTERLEAVED)
                        a0 = a0 + ev
                        a1 = a1 + od
                    return a0, a1

                z = jnp.zeros((LN,), jnp.float32)
                a0, a1 = lax.fori_loop(0, L // 8, acc_body, (z, z))
                q = r // 4
                o = (r - 4 * q) * (2 * LN)
                pooled_v[q, pl.ds(o, LN)] = a0
                pooled_v[q, pl.ds(o + LN, LN)] = a1
                return 0

            lax.fori_loop(0, G, row_body, 0)
            pltpu.sync_copy(pooled_v,
                            out_hbm.at[pl.ds((base_row + c * G) // 4, G // 4)])

        fetch(0, idx_a, sem_a, rows_a)

        def pair_body(c2, _):
            c = 2 * c2
            fetch(c + 1, idx_b, sem_b, rows_b)
            drain_and_acc(c, idx_a, sem_a, rows_a)

            @pl.when(c + 2 < n_chunks)
            def _():
                fetch(c + 2, idx_a, sem_a, rows_a)

            drain_and_acc(c + 1, idx_b, sem_b, rows_b)
            return 0

        lax.fori_loop(0, n_chunks // 2, pair_body, 0)

    return sc


def _mlp_body(x_ref, w_ref, b_ref, o_ref):
    o_ref[...] = jnp.dot(x_ref[...], w_ref[...],
                         preferred_element_type=jnp.float32) + b_ref[...]


def _mlp(pooled, wcp, bcp):
    R, H4 = pooled.shape
    H = H4 // 4
    blk = 4096
    assert R % blk == 0
    wbig = jnp.einsum('ij,kl->ikjl', jnp.eye(4, dtype=jnp.float32),
                      wcp).reshape(4 * H, 4 * DP)
    bbig = jnp.tile(bcp, (1, 4))
    return pl.pallas_call(
        _mlp_body,
        grid=(R // blk,),
        in_specs=[pl.BlockSpec((blk, 4 * H), lambda i: (i, 0)),
                  pl.BlockSpec((4 * H, 4 * DP), lambda i: (0, 0)),
                  pl.BlockSpec((1, 4 * DP), lambda i: (0, 0))],
        out_specs=pl.BlockSpec((blk, 4 * DP), lambda i: (i, 0)),
        out_shape=jax.ShapeDtypeStruct((R, 4 * DP), jnp.float32),
    )(pooled, wbig, bbig)


def kernel(text, text_lengths, emb, W1, b1, W2, b2):
    B, L = text.shape
    V, H = emb.shape
    ncls = W2.shape[1]
    wc = (W1 @ W2) * (1.0 / L)
    perm = jnp.arange(H).reshape(H // 2, 2).T.reshape(H)
    wcp = jnp.zeros((H, DP), jnp.float32).at[:, :ncls].set(wc[perm])
    bcp = jnp.zeros((1, DP), jnp.float32).at[0, :ncls].set(b1 @ W2 + b2)
    embh = emb.astype(jnp.bfloat16)
    text_flat = text.reshape(-1).astype(jnp.int32)
    pooled = _make_sc(B, L, H)(text_flat, embh)
    return _mlp(pooled, wcp, bcp).reshape(B, DP)[:, :ncls]


# speedup vs baseline: 1.0043x; 1.0043x over previous
import functools

import jax
import jax.numpy as jnp
from jax import lax
from jax.experimental import pallas as pl
from jax.experimental.pallas import tpu as pltpu
from jax.experimental.pallas import tpu_sc as plsc

DP = 16
NC, NS, LN = 2, 16, 16


@functools.cache
def _make_sc(B, L, H):
    NW = NC * NS
    rows_per_w = B // NW
    G = 16
    CH = G * L
    n_chunks = rows_per_w // G
    assert B % NW == 0 and rows_per_w % (2 * G) == 0 and L % 8 == 0
    assert H == 2 * LN
    mesh = plsc.VectorSubcoreMesh(core_axis_name="c", subcore_axis_name="s")

    @functools.partial(
        pl.kernel,
        out_type=jax.ShapeDtypeStruct((B // 4, 4 * H), jnp.float32),
        mesh=mesh,
        compiler_params=pltpu.CompilerParams(use_tc_tiling_on_sc=False,
                                             needs_layout_passes=False),
        scratch_types=[
            pltpu.VMEM((CH,), jnp.int32),
            pltpu.VMEM((CH,), jnp.int32),
            pltpu.VMEM((CH, H), jnp.bfloat16),
            pltpu.VMEM((CH, H), jnp.bfloat16),
            pltpu.VMEM((rows_per_w // 4, 4 * H), jnp.float32),
            pltpu.SemaphoreType.DMA,
            pltpu.SemaphoreType.DMA,
        ],
    )
    def sc(text_hbm, emb_hbm, out_hbm, idx_a, idx_b, rows_a, rows_b,
           pooled_v, sem_a, sem_b):
        wid = lax.axis_index("s") * NC + lax.axis_index("c")
        base_row = wid * rows_per_w

        def fetch(c, idx_v, sem, rows_v):
            pltpu.sync_copy(text_hbm.at[pl.ds((base_row + c * G) * L, CH)],
                            idx_v)
            pltpu.async_copy(emb_hbm.at[idx_v], rows_v, sem)

        def drain_and_acc(c, idx_v, sem, rows_v):
            pltpu.make_async_copy(emb_hbm.at[idx_v], rows_v, sem).wait()

            def row_body(r, _):
                def acc_body(i, carry):
                    a0, a1 = carry
                    t = r * L + i * 8
                    for u in range(8):
                        ev, od = plsc.unpack(
                            rows_v[t + u], format=plsc.PackFormat.INTERLEAVED)
                        a0 = a0 + ev
                        a1 = a1 + od
                    return a0, a1

                z = jnp.zeros((LN,), jnp.float32)
                a0, a1 = lax.fori_loop(0, L // 8, acc_body, (z, z))
                g = c * G + r
                q = g // 4
                o = (g - 4 * q) * (2 * LN)
                pooled_v[q, pl.ds(o, LN)] = a0
                pooled_v[q, pl.ds(o + LN, LN)] = a1
                return 0

            lax.fori_loop(0, G, row_body, 0)

        fetch(0, idx_a, sem_a, rows_a)

        def pair_body(c2, _):
            c = 2 * c2
            fetch(c + 1, idx_b, sem_b, rows_b)
            drain_and_acc(c, idx_a, sem_a, rows_a)

            @pl.when(c + 2 < n_chunks)
            def _():
                fetch(c + 2, idx_a, sem_a, rows_a)

            drain_and_acc(c + 1, idx_b, sem_b, rows_b)
            return 0

        lax.fori_loop(0, n_chunks // 2, pair_body, 0)
        pltpu.sync_copy(pooled_v,
                        out_hbm.at[pl.ds(base_row // 4, rows_per_w // 4)])

    return sc


def _mlp_body(x_ref, w_ref, b_ref, o_ref):
    o_ref[...] = jnp.dot(x_ref[...], w_ref[...],
                         preferred_element_type=jnp.float32) + b_ref[...]


def _mlp(pooled, wcp, bcp):
    R, H4 = pooled.shape
    H = H4 // 4
    blk = 4096
    assert R % blk == 0
    wbig = jnp.einsum('ij,kl->ikjl', jnp.eye(4, dtype=jnp.float32),
                      wcp).reshape(4 * H, 4 * DP)
    bbig = jnp.tile(bcp, (1, 4))
    return pl.pallas_call(
        _mlp_body,
        grid=(R // blk,),
        in_specs=[pl.BlockSpec((blk, 4 * H), lambda i: (i, 0)),
                  pl.BlockSpec((4 * H, 4 * DP), lambda i: (0, 0)),
                  pl.BlockSpec((1, 4 * DP), lambda i: (0, 0))],
        out_specs=pl.BlockSpec((blk, 4 * DP), lambda i: (i, 0)),
        out_shape=jax.ShapeDtypeStruct((R, 4 * DP), jnp.float32),
    )(pooled, wbig, bbig)


def kernel(text, text_lengths, emb, W1, b1, W2, b2):
    B, L = text.shape
    V, H = emb.shape
    ncls = W2.shape[1]
    wc = (W1 @ W2) * (1.0 / L)
    perm = jnp.arange(H).reshape(H // 2, 2).T.reshape(H)
    wcp = jnp.zeros((H, DP), jnp.float32).at[:, :ncls].set(wc[perm])
    bcp = jnp.zeros((1, DP), jnp.float32).at[0, :ncls].set(b1 @ W2 + b2)
    embh = emb.astype(jnp.bfloat16)
    text_flat = text.reshape(-1).astype(jnp.int32)
    pooled = _make_sc(B, L, H)(text_flat, embh)
    return _mlp(pooled, wcp, bcp).reshape(B, DP)[:, :ncls]
